# Initial kernel scaffold; baseline (speedup 1.0000x reference)
import jax
import jax.numpy as jnp
from jax.experimental import pallas as pl


def kernel(edge_index, edge_feat, node_emb, weight):
    raise NotImplementedError("write your pallas kernel here")


# speedup vs baseline: 1.6892x; 1.6892x over previous
import jax
import jax.numpy as jnp
from jax import lax
from jax.experimental import pallas as pl
from jax.experimental.pallas import tpu as pltpu
from jax.experimental.pallas import tpu_sc as plsc

_N = 10000
_E = 320000
_D = 128
_DE = 16
_C = 2
_W272 = 2 * _D + _DE

_NC = 2
_NS = 16
_NW = _NC * _NS
_PER_W = _E // _NW
_CH = 128
_NFULL = _PER_W // _CH
_REM = _PER_W - _NFULL * _CH
_L = 16


def _sc_body(src_hbm, dst_hbm, emb_hbm, ef_hbm, nproj_hbm, eproj_hbm,
             out_hbm, sco_hbm,
             idx_s, idx_d, rows_s, rows_d, efb, scb,
             idx_s2, idx_d2, rows_s2, rows_d2, efb2, scb2,
             nsv, sem_a, sem_b, sem_w0, sem_w1, sem_w2):
  wid = lax.axis_index("s") * _NC + lax.axis_index("c")
  pltpu.sync_copy(nproj_hbm, nsv)
  lanes = lax.broadcasted_iota(jnp.int32, (_L,), 0)

  def chunk(base, n, ixs, ixd, rws, rwd, efv, scv):
    pltpu.sync_copy(src_hbm.at[pl.ds(base, n)], ixs)
    pltpu.sync_copy(dst_hbm.at[pl.ds(base, n)], ixd)
    g1 = pltpu.async_copy(emb_hbm.at[ixs], rws, sem_a)
    g2 = pltpu.async_copy(emb_hbm.at[ixd], rwd, sem_b)
    pltpu.sync_copy(ef_hbm.at[pl.ds(base, n)], efv)
    w3 = pltpu.async_copy(efv, out_hbm.at[pl.ds(base, n), pl.ds(2 * _D, _DE)],
                          sem_w2)
    pltpu.sync_copy(eproj_hbm.at[pl.ds(2 * base, 2 * n)], scv)
    for i in range(n // _L):
      si = ixs[pl.ds(i * _L, _L)] * 4
      di = ixd[pl.ds(i * _L, _L)] * 4
      v0 = plsc.load_gather(nsv, [si]) + plsc.load_gather(nsv, [di + 2])
      v1 = plsc.load_gather(nsv, [si + 1]) + plsc.load_gather(nsv, [di + 3])
      p0 = (lanes + i * _L) * 2
      plsc.addupdate_scatter(scv, [p0], v0)
      plsc.addupdate_scatter(scv, [p0 + 1], v1)
    pltpu.sync_copy(scv, sco_hbm.at[pl.ds(2 * base, 2 * n)])
    g1.wait()
    w1 = pltpu.async_copy(rws, out_hbm.at[pl.ds(base, n), pl.ds(0, _D)], sem_w0)
    g2.wait()
    w2 = pltpu.async_copy(rwd, out_hbm.at[pl.ds(base, n), pl.ds(_D, _D)], sem_w1)
    w1.wait()
    w2.wait()
    w3.wait()

  def body(t, carry):
    chunk(wid * _PER_W + t * _CH, _CH, idx_s, idx_d, rows_s, rows_d, efb, scb)
    return carry

  lax.fori_loop(0, _NFULL, body, 0)
  if _REM:
    chunk(wid * _PER_W + _NFULL * _CH, _REM,
          idx_s2, idx_d2, rows_s2, rows_d2, efb2, scb2)


def _dot_body(x_ref, w_ref, o_ref):
  o_ref[...] = jnp.dot(x_ref[...], w_ref[...],
                       preferred_element_type=jnp.float32)


def kernel(edge_index, edge_feat, node_emb, weight):
  src = edge_index[0]
  dst = edge_index[1]
  w_nodes = jnp.concatenate([weight[:, :_D].T, weight[:, _D:2 * _D].T], axis=1)
  w_edge = weight[:, 2 * _D:].T

  node_proj = pl.pallas_call(
      _dot_body,
      out_shape=jax.ShapeDtypeStruct((_N, 2 * _C), jnp.float32),
  )(node_emb, w_nodes)

  _EB = 8000
  ef_proj = pl.pallas_call(
      _dot_body,
      grid=(_E // _EB,),
      in_specs=[pl.BlockSpec((_EB, _DE), lambda i: (i, 0)),
                pl.BlockSpec((_DE, _C), lambda i: (0, 0))],
      out_specs=pl.BlockSpec((_EB, _C), lambda i: (i, 0)),
      out_shape=jax.ShapeDtypeStruct((_E, _C), jnp.float32),
  )(edge_feat, w_edge)

  sc_fn = pl.kernel(
      _sc_body,
      out_type=(jax.ShapeDtypeStruct((_E, _W272), jnp.float32),
                jax.ShapeDtypeStruct((2 * _E,), jnp.float32)),
      mesh=plsc.VectorSubcoreMesh(core_axis_name="c", subcore_axis_name="s",
                                  num_cores=_NC, num_subcores=_NS),
      compiler_params=pltpu.CompilerParams(needs_layout_passes=False),
      scratch_types=[
          pltpu.VMEM((_CH,), jnp.int32),
          pltpu.VMEM((_CH,), jnp.int32),
          pltpu.VMEM((_CH, _D), jnp.float32),
          pltpu.VMEM((_CH, _D), jnp.float32),
          pltpu.VMEM((_CH, _DE), jnp.float32),
          pltpu.VMEM((2 * _CH,), jnp.float32),
          pltpu.VMEM((_REM,), jnp.int32),
          pltpu.VMEM((_REM,), jnp.int32),
          pltpu.VMEM((_REM, _D), jnp.float32),
          pltpu.VMEM((_REM, _D), jnp.float32),
          pltpu.VMEM((_REM, _DE), jnp.float32),
          pltpu.VMEM((2 * _REM,), jnp.float32),
          pltpu.VMEM((_N * 2 * _C,), jnp.float32),
          pltpu.SemaphoreType.DMA,
          pltpu.SemaphoreType.DMA,
          pltpu.SemaphoreType.DMA,
          pltpu.SemaphoreType.DMA,
          pltpu.SemaphoreType.DMA,
      ],
  )
  edge_embeds, scores_flat = sc_fn(src, dst, node_emb, edge_feat,
                                   node_proj.reshape(-1), ef_proj.reshape(-1))
  return scores_flat.reshape(_E, _C), edge_embeds


# speedup vs baseline: 1.8495x; 1.0949x over previous
import jax
import jax.numpy as jnp
from jax import lax
from jax.experimental import pallas as pl
from jax.experimental.pallas import tpu as pltpu
from jax.experimental.pallas import tpu_sc as plsc

_N = 10000
_E = 320000
_D = 128
_DE = 16
_C = 2
_W272 = 2 * _D + _DE

_NC = 2
_NS = 16
_NW = _NC * _NS
_PER_W = _E // _NW
_CH = 80
_NFULL = _PER_W // _CH
_L = 16


def _sc_body(src_hbm, dst_hbm, emb_hbm, ef_hbm, nproj_hbm, eproj_hbm,
             out_hbm, sco_hbm,
             ixs0, ixd0, ixs1, ixd1, rws0, rws1, rwd0, rwd1, efv0, efv1, scv0, scv1,
             nsv,
             six0, six1, sgs0, sgs1, sgd0, sgd1, sef0, sef1, ssc0, ssc1,
             sw0, sw1):
  wid = lax.axis_index("s") * _NC + lax.axis_index("c")
  pltpu.sync_copy(nproj_hbm, nsv)
  lanes = lax.broadcasted_iota(jnp.int32, (_L,), 0)
  w0 = wid * _PER_W

  bufs = ((ixs0, ixd0, rws0, rwd0, efv0, scv0, six0, sgs0, sgd0, sef0, ssc0, sw0),
          (ixs1, ixd1, rws1, rwd1, efv1, scv1, six1, sgs1, sgd1, sef1, ssc1, sw1))

  def idx_descs(t, bs):
    ixs, ixd, six = bs[0], bs[1], bs[6]
    base = w0 + t * _CH
    return (pltpu.make_async_copy(src_hbm.at[pl.ds(base, _CH)], ixs, six),
            pltpu.make_async_copy(dst_hbm.at[pl.ds(base, _CH)], ixd, six))

  def issue(t, bs):
    ixs, ixd, rws, rwd, efv, scv, _, sgs, sgd, sef, ssc, _ = bs
    base = w0 + t * _CH
    pltpu.async_copy(emb_hbm.at[ixs], rws, sgs)
    pltpu.async_copy(emb_hbm.at[ixd], rwd, sgd)
    pltpu.async_copy(ef_hbm.at[pl.ds(base, _CH)], efv, sef)
    pltpu.async_copy(eproj_hbm.at[pl.ds(2 * base, 2 * _CH)], scv, ssc)

  def scores(ixs, ixd, scv, n):
    for i in range(n // _L):
      si = ixs[pl.ds(i * _L, _L)] * 4
      di = ixd[pl.ds(i * _L, _L)] * 4
      v0 = plsc.load_gather(nsv, [si]) + plsc.load_gather(nsv, [di + 2])
      v1 = plsc.load_gather(nsv, [si + 1]) + plsc.load_gather(nsv, [di + 3])
      p0 = (lanes + i * _L) * 2
      plsc.addupdate_scatter(scv, [p0], v0)
      plsc.addupdate_scatter(scv, [p0 + 1], v1)

  def write_descs(t, bs):
    _, _, rws, rwd, efv, scv, _, _, _, _, _, sw = bs
    base = w0 + t * _CH
    return (
        pltpu.make_async_copy(rws, out_hbm.at[pl.ds(base, _CH), pl.ds(0, _D)], sw),
        pltpu.make_async_copy(rwd, out_hbm.at[pl.ds(base, _CH), pl.ds(_D, _D)], sw),
        pltpu.make_async_copy(efv, out_hbm.at[pl.ds(base, _CH), pl.ds(2 * _D, _DE)], sw),
        pltpu.make_async_copy(scv, sco_hbm.at[pl.ds(2 * base, 2 * _CH)], sw),
    )

  def finish(t, bs):
    ixs, ixd, rws, rwd, efv, scv, _, sgs, sgd, sef, ssc, sw = bs
    base = w0 + t * _CH
    pltpu.make_async_copy(
        eproj_hbm.at[pl.ds(2 * base, 2 * _CH)], scv, ssc).wait()
    scores(ixs, ixd, scv, _CH)
    d1, d2, d3, d4 = write_descs(t, bs)
    pltpu.make_async_copy(emb_hbm.at[ixs], rws, sgs).wait()
    d1.start()
    pltpu.make_async_copy(emb_hbm.at[ixd], rwd, sgd).wait()
    d2.start()
    pltpu.make_async_copy(ef_hbm.at[pl.ds(base, _CH)], efv, sef).wait()
    d3.start()
    d4.start()

  def drain(t, bs):
    for d in write_descs(t, bs):
      d.wait()

  for d in idx_descs(0, bufs[0]):
    d.start()

  def body(s, carry):
    for b in (0, 1):
      t = s * 2 + b
      cur, prev = bufs[b], bufs[1 - b]

      @pl.when(t >= 1)
      def _():
        finish(t - 1, prev)

      @pl.when(t >= 2)
      def _():
        drain(t - 2, cur)

      for d in idx_descs(t, cur):
        d.wait()
      issue(t, cur)

      @pl.when(t < _NFULL - 1)
      def _():
        for d in idx_descs(t + 1, prev):
          d.start()
    return carry

  lax.fori_loop(0, _NFULL // 2, body, 0)
  if _NFULL % 2:
    t_last = _NFULL - 1
    finish(t_last - 1, bufs[1])
    drain(t_last - 2, bufs[0])
    for d in idx_descs(t_last, bufs[0]):
      d.wait()
    issue(t_last, bufs[0])
    finish(t_last, bufs[0])
    drain(t_last - 1, bufs[1])
    drain(t_last, bufs[0])
  else:
    finish(_NFULL - 1, bufs[1])
    drain(_NFULL - 2, bufs[0])
    drain(_NFULL - 1, bufs[1])


def _dot_body(x_ref, w_ref, o_ref):
  o_ref[...] = jnp.dot(x_ref[...], w_ref[...],
                       preferred_element_type=jnp.float32)


def kernel(edge_index, edge_feat, node_emb, weight):
  w_nodes = jnp.concatenate([weight[:, :_D].T, weight[:, _D:2 * _D].T], axis=1)
  w_edge = weight[:, 2 * _D:].T

  node_proj = pl.pallas_call(
      _dot_body,
      out_shape=jax.ShapeDtypeStruct((_N, 2 * _C), jnp.float32),
  )(node_emb, w_nodes)

  _EB = 8000
  ef_proj = pl.pallas_call(
      _dot_body,
      grid=(_E // _EB,),
      in_specs=[pl.BlockSpec((_EB, _DE), lambda i: (i, 0)),
                pl.BlockSpec((_DE, _C), lambda i: (0, 0))],
      out_specs=pl.BlockSpec((_EB, _C), lambda i: (i, 0)),
      out_shape=jax.ShapeDtypeStruct((_E, _C), jnp.float32),
  )(edge_feat, w_edge)

  sc_fn = pl.kernel(
      _sc_body,
      out_type=(jax.ShapeDtypeStruct((_E, _W272), jnp.float32),
                jax.ShapeDtypeStruct((2 * _E,), jnp.float32)),
      mesh=plsc.VectorSubcoreMesh(core_axis_name="c", subcore_axis_name="s",
                                  num_cores=_NC, num_subcores=_NS),
      compiler_params=pltpu.CompilerParams(needs_layout_passes=False),
      scratch_types=[
          pltpu.VMEM((_CH,), jnp.int32),
          pltpu.VMEM((_CH,), jnp.int32),
          pltpu.VMEM((_CH,), jnp.int32),
          pltpu.VMEM((_CH,), jnp.int32),
          pltpu.VMEM((_CH, _D), jnp.float32),
          pltpu.VMEM((_CH, _D), jnp.float32),
          pltpu.VMEM((_CH, _D), jnp.float32),
          pltpu.VMEM((_CH, _D), jnp.float32),
          pltpu.VMEM((_CH, _DE), jnp.float32),
          pltpu.VMEM((_CH, _DE), jnp.float32),
          pltpu.VMEM((2 * _CH,), jnp.float32),
          pltpu.VMEM((2 * _CH,), jnp.float32),
          pltpu.VMEM((_N * 2 * _C,), jnp.float32),
          pltpu.SemaphoreType.DMA,
          pltpu.SemaphoreType.DMA,
          pltpu.SemaphoreType.DMA,
          pltpu.SemaphoreType.DMA,
          pltpu.SemaphoreType.DMA,
          pltpu.SemaphoreType.DMA,
          pltpu.SemaphoreType.DMA,
          pltpu.SemaphoreType.DMA,
          pltpu.SemaphoreType.DMA,
          pltpu.SemaphoreType.DMA,
          pltpu.SemaphoreType.DMA,
          pltpu.SemaphoreType.DMA,
      ],
  )
  edge_embeds, scores_flat = sc_fn(edge_index[0], edge_index[1], node_emb,
                                   edge_feat, node_proj.reshape(-1),
                                   ef_proj.reshape(-1))
  return scores_flat.reshape(_E, _C), edge_embeds
